# baseline (speedup 1.0000x reference)
import functools

import jax
import jax.numpy as jnp
from jax import lax
from jax.experimental import pallas as pl
from jax.experimental.pallas import tpu as pltpu
from jax.experimental.pallas import tpu_sc as plsc

_K = 1024
_D = 64
_B = 16
_HW = 1024
_N = _B * _HW

_L = 16
_P = 256
_BSC = 8


def _vq_full_body(x_ref, cb_ref, idx_ref, emb_ref):
    _argmin_common(x_ref, cb_ref, idx_ref, emb_ref)


def _argmin_body(x_ref, cb_ref, idx_ref):
    _argmin_common(x_ref, cb_ref, idx_ref, None)


def _argmin_common(x_ref, cb_ref, idx_ref, emb_ref):
    xT = x_ref[0]
    cb = cb_ref[...]
    scoresT = lax.dot_general(cb, xT, (((1,), (0,)), ((), ())),
                              preferred_element_type=jnp.float32)
    x2 = jnp.sum(xT * xT, axis=0, keepdims=True)
    c2 = jnp.sum(cb * cb, axis=1, keepdims=True)
    d2 = (x2 + c2) - 2.0 * scoresT
    m2 = jnp.min(d2, axis=0, keepdims=True)
    m2c = jnp.maximum(m2, 0.0)
    s = jnp.sqrt(m2c)
    hb = lax.bitcast_convert_type(s * s, jnp.int32)
    hi = m2c
    for k in range(-4, 5):
        c = lax.bitcast_convert_type(hb + k, jnp.float32)
        ok = (c >= 0.0) & (jnp.sqrt(c) == s)
        hi = jnp.where(ok, jnp.maximum(hi, c), hi)
    hi = jnp.where(s > 0.0, hi, 0.0)
    kiota = lax.broadcasted_iota(jnp.int32, (_K, _HW), 0)
    idx = jnp.min(jnp.where(d2 <= hi, kiota, _K), axis=0)
    idx_ref[0] = idx.reshape(1, _HW)
    if emb_ref is not None:
        onehotT = (kiota == idx[None, :]).astype(jnp.float32)
        embT = lax.dot_general(cb, onehotT, (((0,), (0,)), ((), ())),
                               preferred_element_type=jnp.float32,
                               precision=lax.Precision.HIGHEST)
        emb_ref[0] = embT


def _vq_full(inp, codebook):
    return pl.pallas_call(
        _vq_full_body,
        grid=(_B,),
        in_specs=[
            pl.BlockSpec((1, _D, _HW), lambda b: (b, 0, 0)),
            pl.BlockSpec((_K, _D), lambda b: (0, 0)),
        ],
        out_specs=[
            pl.BlockSpec((1, 1, _HW), lambda b: (b, 0, 0)),
            pl.BlockSpec((1, _D, _HW), lambda b: (b, 0, 0)),
        ],
        out_shape=[
            jax.ShapeDtypeStruct((_B, 1, _HW), jnp.int32),
            jax.ShapeDtypeStruct((_B, _D, _HW), jnp.float32),
        ],
    )(inp, codebook)


def _compute_idx(inp, codebook):
    return pl.pallas_call(
        _argmin_body,
        grid=(_B,),
        in_specs=[
            pl.BlockSpec((1, _D, _HW), lambda b: (b, 0, 0)),
            pl.BlockSpec((_K, _D), lambda b: (0, 0)),
        ],
        out_specs=pl.BlockSpec((1, 1, _HW), lambda b: (b, 0, 0)),
        out_shape=jax.ShapeDtypeStruct((_B, 1, _HW), jnp.int32),
    )(inp, codebook)


@functools.partial(
    pl.kernel,
    mesh=plsc.VectorSubcoreMesh(core_axis_name="c", subcore_axis_name="s"),
    compiler_params=pltpu.CompilerParams(needs_layout_passes=False,
                                         use_tc_tiling_on_sc=False),
    out_type=jax.ShapeDtypeStruct((_BSC * _D * 4, _P), jnp.float32),
    scratch_types=[
        pltpu.VMEM((2, 128), jnp.int32),
        pltpu.VMEM((_P, _D), jnp.float32),
        pltpu.VMEM((_P * (_D + 1),), jnp.float32),
        pltpu.VMEM((_D, _P), jnp.float32),
        pltpu.VMEM((_D,), jnp.int32),
        pltpu.SemaphoreType.DMA,
        pltpu.SemaphoreType.DMA,
    ],
)
def _sc_gather(cb_hbm, idx_hbm, out_hbm, idx_v, rows_v, rows_p, out_v, oidx_v,
               sem1, sem2):
    wid = lax.axis_index("s") * 2 + lax.axis_index("c")
    b = wid // 4
    quarter = wid % 4
    lane = lax.iota(jnp.int32, _L)
    cps = [
        pltpu.async_copy(idx_hbm.at[pl.ds(wid * _P + j * 128, 128)],
                         idx_v.at[j], sem1)
        for j in range(2)
    ]
    for q in range(_D // _L):
        oidx_v[pl.ds(q * _L, _L)] = (
            b * (_D * 4) + quarter + (q * _L + lane) * 4)
    for cp in cps:
        cp.wait()
    cps = [
        pltpu.async_copy(cb_hbm.at[idx_v.at[j]],
                         rows_v.at[pl.ds(j * 128, 128), :], sem2)
        for j in range(2)
    ]
    for cp in cps:
        cp.wait()

    def repitch(n, carry):
        for q in range(_D // _L):
            rows_p[pl.ds(n * (_D + 1) + q * _L, _L)] = (
                rows_v[n, pl.ds(q * _L, _L)])
        return carry

    lax.fori_loop(0, _P, repitch, 0)

    def body(g, carry):
        prow = g * _L + lane
        for d in range(_D):
            out_v[d, pl.ds(g * _L, _L)] = plsc.load_gather(
                rows_p, [prow * (_D + 1) + d])
        return carry

    lax.fori_loop(0, _P // _L, body, 0)
    pltpu.sync_copy(out_v, out_hbm.at[oidx_v])


def _onehot_body(idx_ref, cb_ref, emb_ref):
    idx = idx_ref[...].reshape(_HW)
    cb = cb_ref[...]
    kiota = lax.broadcasted_iota(jnp.int32, (_K, _HW), 0)
    onehotT = (kiota == idx[None, :]).astype(jnp.float32)
    embT = lax.dot_general(cb, onehotT, (((0,), (0,)), ((), ())),
                           preferred_element_type=jnp.float32,
                           precision=lax.Precision.HIGHEST)
    emb_ref[0] = embT


def _tc_gather_hi(idx3, codebook):
    return pl.pallas_call(
        _onehot_body,
        grid=(_B - _BSC,),
        in_specs=[
            pl.BlockSpec((1, 1, _HW), lambda b: (b + _BSC, 0, 0)),
            pl.BlockSpec((_K, _D), lambda b: (0, 0)),
        ],
        out_specs=pl.BlockSpec((1, _D, _HW), lambda b: (b, 0, 0)),
        out_shape=jax.ShapeDtypeStruct((_B - _BSC, _D, _HW), jnp.float32),
    )(idx3, codebook)


def kernel(input, codebook):
    inp = input.reshape(_B, _D, _HW)
    idx3 = _compute_idx(inp, codebook)
    emb_sc = _sc_gather(codebook, idx3.reshape(_N))
    emb_tc = _tc_gather_hi(idx3, codebook)
    emb3 = jnp.concatenate([emb_sc.reshape(_BSC, _D, _HW), emb_tc], axis=0)
    embed = emb3.reshape(_B, _D, 32, 32)
    idxes = idx3.reshape(_B, 32, 32)
    return (embed, idxes)


# speedup vs baseline: 1.4170x; 1.4170x over previous
import functools

import jax
import jax.numpy as jnp
from jax import lax
from jax.experimental import pallas as pl
from jax.experimental.pallas import tpu as pltpu
from jax.experimental.pallas import tpu_sc as plsc

_K = 1024
_D = 64
_B = 16
_HW = 1024
_N = _B * _HW

_L = 16
_P = 256
_BSC = 8


def _vq_full_body(x_ref, cb_ref, idx_ref, emb_ref):
    _argmin_common(x_ref, cb_ref, idx_ref, emb_ref)


def _argmin_body(x_ref, cb_ref, idx_ref):
    _argmin_common(x_ref, cb_ref, idx_ref, None)


def _argmin_common(x_ref, cb_ref, idx_ref, emb_ref):
    xT = x_ref[0]
    cb = cb_ref[...]
    scoresT = lax.dot_general(cb, xT, (((1,), (0,)), ((), ())),
                              preferred_element_type=jnp.float32)
    x2 = jnp.sum(xT * xT, axis=0, keepdims=True)
    c2 = jnp.sum(cb * cb, axis=1, keepdims=True)
    d2 = (x2 + c2) - 2.0 * scoresT
    m2 = jnp.min(d2, axis=0, keepdims=True)
    m2c = jnp.maximum(m2, 0.0)
    s = jnp.sqrt(m2c)
    hb = lax.bitcast_convert_type(s * s, jnp.int32)
    hi = m2c
    for k in range(-4, 5):
        c = lax.bitcast_convert_type(hb + k, jnp.float32)
        ok = (c >= 0.0) & (jnp.sqrt(c) == s)
        hi = jnp.where(ok, jnp.maximum(hi, c), hi)
    hi = jnp.where(s > 0.0, hi, 0.0)
    kiota = lax.broadcasted_iota(jnp.int32, (_K, _HW), 0)
    idx = jnp.min(jnp.where(d2 <= hi, kiota, _K), axis=0)
    idx_ref[0] = idx.reshape(1, _HW)
    if emb_ref is not None:
        onehotT = (kiota == idx[None, :]).astype(jnp.float32)
        embT = lax.dot_general(cb, onehotT, (((0,), (0,)), ((), ())),
                               preferred_element_type=jnp.float32,
                               precision=lax.Precision.HIGHEST)
        emb_ref[0] = embT


def _vq_full(inp, codebook):
    return pl.pallas_call(
        _vq_full_body,
        grid=(_B,),
        in_specs=[
            pl.BlockSpec((1, _D, _HW), lambda b: (b, 0, 0)),
            pl.BlockSpec((_K, _D), lambda b: (0, 0)),
        ],
        out_specs=[
            pl.BlockSpec((1, 1, _HW), lambda b: (b, 0, 0)),
            pl.BlockSpec((1, _D, _HW), lambda b: (b, 0, 0)),
        ],
        out_shape=[
            jax.ShapeDtypeStruct((_B, 1, _HW), jnp.int32),
            jax.ShapeDtypeStruct((_B, _D, _HW), jnp.float32),
        ],
    )(inp, codebook)


def _compute_idx(inp, codebook):
    return pl.pallas_call(
        _argmin_body,
        grid=(_B,),
        in_specs=[
            pl.BlockSpec((1, _D, _HW), lambda b: (b, 0, 0)),
            pl.BlockSpec((_K, _D), lambda b: (0, 0)),
        ],
        out_specs=pl.BlockSpec((1, 1, _HW), lambda b: (b, 0, 0)),
        out_shape=jax.ShapeDtypeStruct((_B, 1, _HW), jnp.int32),
    )(inp, codebook)


@functools.partial(
    pl.kernel,
    mesh=plsc.VectorSubcoreMesh(core_axis_name="c", subcore_axis_name="s"),
    compiler_params=pltpu.CompilerParams(needs_layout_passes=False,
                                         use_tc_tiling_on_sc=False),
    out_type=jax.ShapeDtypeStruct((_BSC * _D * 4, _P), jnp.float32),
    scratch_types=[
        pltpu.VMEM((2, 128), jnp.int32),
        pltpu.VMEM((_P, _D), jnp.float32),
        pltpu.VMEM((_P * (_D + 1),), jnp.float32),
        pltpu.VMEM((_D, _P), jnp.float32),
        pltpu.VMEM((_D,), jnp.int32),
        pltpu.SemaphoreType.DMA,
        pltpu.SemaphoreType.DMA,
    ],
)
def _sc_gather(cb_hbm, idx_hbm, out_hbm, idx_v, rows_v, rows_p, out_v, oidx_v,
               sem1, sem2):
    wid = lax.axis_index("s") * 2 + lax.axis_index("c")
    b = wid // 4
    quarter = wid % 4
    lane = lax.iota(jnp.int32, _L)
    cps = [
        pltpu.async_copy(idx_hbm.at[pl.ds(wid * _P + j * 128, 128)],
                         idx_v.at[j], sem1)
        for j in range(2)
    ]
    for q in range(_D // _L):
        oidx_v[pl.ds(q * _L, _L)] = (
            b * (_D * 4) + quarter + (q * _L + lane) * 4)
    for cp in cps:
        cp.wait()
    cps = [
        pltpu.async_copy(cb_hbm.at[idx_v.at[j]],
                         rows_v.at[pl.ds(j * 128, 128), :], sem2)
        for j in range(2)
    ]
    for cp in cps:
        cp.wait()

    def repitch(n, carry):
        for q in range(_D // _L):
            rows_p[pl.ds(n * (_D + 1) + q * _L, _L)] = (
                rows_v[n, pl.ds(q * _L, _L)])
        return carry

    lax.fori_loop(0, _P, repitch, 0)

    def body(g, carry):
        prow = g * _L + lane
        for d in range(_D):
            out_v[d, pl.ds(g * _L, _L)] = plsc.load_gather(
                rows_p, [prow * (_D + 1) + d])
        return carry

    lax.fori_loop(0, _P // _L, body, 0)
    pltpu.sync_copy(out_v, out_hbm.at[oidx_v])


def _onehot_body(idx_ref, cb_ref, emb_ref):
    idx = idx_ref[...].reshape(_HW)
    cb = cb_ref[...]
    kiota = lax.broadcasted_iota(jnp.int32, (_K, _HW), 0)
    onehotT = (kiota == idx[None, :]).astype(jnp.float32)
    embT = lax.dot_general(cb, onehotT, (((0,), (0,)), ((), ())),
                           preferred_element_type=jnp.float32,
                           precision=lax.Precision.HIGHEST)
    emb_ref[0] = embT


def _tc_gather_hi(idx3, codebook):
    return pl.pallas_call(
        _onehot_body,
        grid=(_B - _BSC,),
        in_specs=[
            pl.BlockSpec((1, 1, _HW), lambda b: (b + _BSC, 0, 0)),
            pl.BlockSpec((_K, _D), lambda b: (0, 0)),
        ],
        out_specs=pl.BlockSpec((1, _D, _HW), lambda b: (b, 0, 0)),
        out_shape=jax.ShapeDtypeStruct((_B - _BSC, _D, _HW), jnp.float32),
    )(idx3, codebook)


def kernel(input, codebook):
    inp = input.reshape(_B, _D, _HW)
    idx3, emb3 = _vq_full(inp, codebook)
    embed = emb3.reshape(_B, _D, 32, 32)
    idxes = idx3.reshape(_B, 32, 32)
    return (embed, idxes)
